# initial kernel scaffold (unmeasured)
import jax
import jax.numpy as jnp
from jax import lax
from jax.experimental import pallas as pl
from jax.experimental.pallas import tpu as pltpu

M = 4096
N = 4096
K = 8192

BM = 512
BK = 512

NC = 8
CM = M // NC


def _matmul_body(dy_ref, w_ref, out_ref, acc_ref):
    k = pl.program_id(1)

    @pl.when(k == 0)
    def _():
        acc_ref[...] = jnp.zeros_like(acc_ref)

    a = dy_ref[...].astype(jnp.bfloat16)
    b = w_ref[...].astype(jnp.bfloat16)
    acc_ref[...] += lax.dot_general(
        a, b, (((1,), (1,)), ((), ())), preferred_element_type=jnp.float32
    )

    @pl.when(k == pl.num_programs(1) - 1)
    def _():
        out_ref[...] = acc_ref[...].astype(jnp.bfloat16)


def _partial_matmul(dy, w):
    return pl.pallas_call(
        _matmul_body,
        grid=(M // BM, K // BK),
        in_specs=[
            pl.BlockSpec((BM, BK), lambda m, k: (m, k)),
            pl.BlockSpec((N, BK), lambda m, k: (0, k)),
        ],
        out_specs=pl.BlockSpec((BM, N), lambda m, k: (m, 0)),
        out_shape=jax.ShapeDtypeStruct((M, N), jnp.bfloat16),
        scratch_shapes=[pltpu.VMEM((BM, N), jnp.float32)],
        compiler_params=pltpu.CompilerParams(
            dimension_semantics=("arbitrary", "arbitrary"),
        ),
    )(dy, w)


def _comm_body(p_ref, out_ref, recv_ref, send_sem, recv_sem):
    i = pl.program_id(0)
    my_x = lax.axis_index("x")
    my_y = lax.axis_index("y")
    my_z = lax.axis_index("z")
    slot = lax.rem(i, 2)

    rdma = pltpu.make_async_remote_copy(
        src_ref=p_ref,
        dst_ref=recv_ref.at[slot],
        send_sem=send_sem.at[slot],
        recv_sem=recv_sem.at[slot],
        device_id=(my_x, 1 - my_y, my_z),
        device_id_type=pltpu.DeviceIdType.MESH,
    )
    rdma.start()
    rdma.wait()

    out_ref[...] = p_ref[...].astype(jnp.float32) + recv_ref[slot].astype(
        jnp.float32
    )


def kernel(dy, W):
    partial = _partial_matmul(dy, W)
    return pl.pallas_call(
        _comm_body,
        grid=(NC,),
        in_specs=[pl.BlockSpec((CM, N), lambda i: (i, 0))],
        out_specs=pl.BlockSpec((CM, N), lambda i: (i, 0)),
        out_shape=jax.ShapeDtypeStruct((M, N), jnp.float32),
        scratch_shapes=[
            pltpu.VMEM((2, CM, N), jnp.bfloat16),
            pltpu.SemaphoreType.DMA((2,)),
            pltpu.SemaphoreType.DMA((2,)),
        ],
        compiler_params=pltpu.CompilerParams(has_side_effects=True),
    )(partial)


# baseline (device time: 868356 ns/iter reference)
import jax
import jax.numpy as jnp
from jax import lax
from jax.experimental import pallas as pl
from jax.experimental.pallas import tpu as pltpu

M = 4096
N = 4096
K = 8192

BM = 512
BK = 512

NC = 8
CM = M // NC


def _matmul_body(dy_ref, w_ref, out_ref, acc_ref):
    k = pl.program_id(1)

    @pl.when(k == 0)
    def _():
        acc_ref[...] = jnp.zeros_like(acc_ref)

    a = dy_ref[...].astype(jnp.bfloat16)
    b = w_ref[...].astype(jnp.bfloat16)
    acc_ref[...] += lax.dot_general(
        a, b, (((1,), (1,)), ((), ())), preferred_element_type=jnp.float32
    )

    @pl.when(k == pl.num_programs(1) - 1)
    def _():
        out_ref[...] = acc_ref[...].astype(jnp.bfloat16)


def _partial_matmul(dy, w):
    return pl.pallas_call(
        _matmul_body,
        grid=(M // BM, K // BK),
        in_specs=[
            pl.BlockSpec((BM, BK), lambda m, k: (m, k)),
            pl.BlockSpec((N, BK), lambda m, k: (0, k)),
        ],
        out_specs=pl.BlockSpec((BM, N), lambda m, k: (m, 0)),
        out_shape=jax.ShapeDtypeStruct((M, N), jnp.bfloat16),
        scratch_shapes=[pltpu.VMEM((BM, N), jnp.float32)],
        compiler_params=pltpu.CompilerParams(
            dimension_semantics=("arbitrary", "arbitrary"),
            vmem_limit_bytes=100 * 1024 * 1024,
        ),
    )(dy, w)


def _comm_body(p_ref, out_ref, recv_ref, send_sem, recv_sem):
    i = pl.program_id(0)
    my_x = lax.axis_index("x")
    my_y = lax.axis_index("y")
    my_z = lax.axis_index("z")
    slot = lax.rem(i, 2)

    rdma = pltpu.make_async_remote_copy(
        src_ref=p_ref,
        dst_ref=recv_ref.at[slot],
        send_sem=send_sem.at[slot],
        recv_sem=recv_sem.at[slot],
        device_id=(my_x, 1 - my_y, my_z),
        device_id_type=pltpu.DeviceIdType.MESH,
    )
    rdma.start()
    rdma.wait()

    out_ref[...] = p_ref[...].astype(jnp.float32) + recv_ref[slot].astype(
        jnp.float32
    )


def kernel(dy, W):
    partial = _partial_matmul(dy, W)
    return pl.pallas_call(
        _comm_body,
        grid=(NC,),
        in_specs=[pl.BlockSpec((CM, N), lambda i: (i, 0))],
        out_specs=pl.BlockSpec((CM, N), lambda i: (i, 0)),
        out_shape=jax.ShapeDtypeStruct((M, N), jnp.float32),
        scratch_shapes=[
            pltpu.VMEM((2, CM, N), jnp.bfloat16),
            pltpu.SemaphoreType.DMA((2,)),
            pltpu.SemaphoreType.DMA((2,)),
        ],
        compiler_params=pltpu.CompilerParams(
            has_side_effects=True,
            vmem_limit_bytes=100 * 1024 * 1024,
        ),
    )(partial)


# device time: 789498 ns/iter; 1.0999x vs baseline; 1.0999x over previous
import jax
import jax.numpy as jnp
from jax import lax
from jax.experimental import pallas as pl
from jax.experimental.pallas import tpu as pltpu

M = 4096
N = 4096
K = 8192

BM = 2048
BN = 2048
BK = 256

NC = 8
CM = M // NC


def _matmul_body(dy_ref, w_ref, out_ref, acc_ref):
    k = pl.program_id(2)

    @pl.when(k == 0)
    def _():
        acc_ref[...] = jnp.zeros_like(acc_ref)

    a = dy_ref[...].astype(jnp.bfloat16)
    b = w_ref[...].astype(jnp.bfloat16)
    acc_ref[...] += lax.dot_general(
        a, b, (((1,), (1,)), ((), ())), preferred_element_type=jnp.float32
    )

    @pl.when(k == pl.num_programs(2) - 1)
    def _():
        out_ref[...] = acc_ref[...].astype(jnp.bfloat16)


def _partial_matmul(dy, w):
    return pl.pallas_call(
        _matmul_body,
        grid=(M // BM, N // BN, K // BK),
        in_specs=[
            pl.BlockSpec((BM, BK), lambda m, n, k: (m, k)),
            pl.BlockSpec((BN, BK), lambda m, n, k: (n, k)),
        ],
        out_specs=pl.BlockSpec((BM, BN), lambda m, n, k: (m, n)),
        out_shape=jax.ShapeDtypeStruct((M, N), jnp.bfloat16),
        scratch_shapes=[pltpu.VMEM((BM, BN), jnp.float32)],
        compiler_params=pltpu.CompilerParams(
            dimension_semantics=("arbitrary", "arbitrary", "arbitrary"),
            vmem_limit_bytes=60 * 1024 * 1024,
        ),
    )(dy, w)


def _comm_body(p_ref, out_ref, recv_ref, send_sem, recv_sem):
    i = pl.program_id(0)
    my_x = lax.axis_index("x")
    my_y = lax.axis_index("y")
    my_z = lax.axis_index("z")
    slot = lax.rem(i, 2)

    rdma = pltpu.make_async_remote_copy(
        src_ref=p_ref,
        dst_ref=recv_ref.at[slot],
        send_sem=send_sem.at[slot],
        recv_sem=recv_sem.at[slot],
        device_id=(my_x, 1 - my_y, my_z),
        device_id_type=pltpu.DeviceIdType.MESH,
    )
    rdma.start()
    rdma.wait()

    out_ref[...] = p_ref[...].astype(jnp.float32) + recv_ref[slot].astype(
        jnp.float32
    )


def kernel(dy, W):
    partial = _partial_matmul(dy, W)
    return pl.pallas_call(
        _comm_body,
        grid=(NC,),
        in_specs=[pl.BlockSpec((CM, N), lambda i: (i, 0))],
        out_specs=pl.BlockSpec((CM, N), lambda i: (i, 0)),
        out_shape=jax.ShapeDtypeStruct((M, N), jnp.float32),
        scratch_shapes=[
            pltpu.VMEM((2, CM, N), jnp.bfloat16),
            pltpu.SemaphoreType.DMA((2,)),
            pltpu.SemaphoreType.DMA((2,)),
        ],
        compiler_params=pltpu.CompilerParams(
            has_side_effects=True,
            vmem_limit_bytes=100 * 1024 * 1024,
        ),
    )(partial)


# device time: 374006 ns/iter; 2.3218x vs baseline; 2.1109x over previous
import jax
import jax.numpy as jnp
from jax import lax
from jax.experimental import pallas as pl
from jax.experimental.pallas import tpu as pltpu

M = 4096
N = 4096
K = 8192

P = 8
SM = M // P
BK = 256

FWD_ROUNDS = 4
BWD_ROUNDS = 3


def _ring_coords(q, my_y):
    xq = q // 4
    zq = jnp.where(xq == 0, q, 7 - q)
    return (xq, my_y, zq)


def _matmul_body(dy_ref, w_ref, out_ref, acc_ref):
    k = pl.program_id(0)

    @pl.when(k == 0)
    def _():
        acc_ref[...] = jnp.zeros_like(acc_ref)

    a = dy_ref[...].astype(jnp.bfloat16)
    b = w_ref[...].astype(jnp.bfloat16)
    acc_ref[...] += lax.dot_general(
        a, b, (((1,), (1,)), ((), ())), preferred_element_type=jnp.float32
    )

    @pl.when(k == pl.num_programs(0) - 1)
    def _():
        out_ref[...] = acc_ref[...].astype(jnp.bfloat16)


def _strip_matmul(dy_strip, w):
    return pl.pallas_call(
        _matmul_body,
        grid=(K // BK,),
        in_specs=[
            pl.BlockSpec((SM, BK), lambda k: (0, k)),
            pl.BlockSpec((N, BK), lambda k: (0, k)),
        ],
        out_specs=pl.BlockSpec((SM, N), lambda k: (0, 0)),
        out_shape=jax.ShapeDtypeStruct((SM, N), jnp.bfloat16),
        scratch_shapes=[pltpu.VMEM((SM, N), jnp.float32)],
        compiler_params=pltpu.CompilerParams(
            dimension_semantics=("arbitrary",),
            vmem_limit_bytes=60 * 1024 * 1024,
        ),
    )(dy_strip, w)


def _comm_body(
    strip_ref,
    out_ref,
    ybuf,
    gather,
    stage,
    y_send, y_recv,
    fwd_send, fwd_recv,
    bwd_send, bwd_recv,
    copy_sem,
):
    my_x = lax.axis_index("x")
    my_y = lax.axis_index("y")
    my_z = lax.axis_index("z")
    p = jnp.where(my_x == 0, my_z, 7 - my_z)
    right = _ring_coords((p + 1) % P, my_y)
    left = _ring_coords((p - 1) % P, my_y)
    ynbr = (my_x, 1 - my_y, my_z)

    barrier = pltpu.get_barrier_semaphore()
    for nbr in (ynbr, left, right):
        pl.semaphore_signal(
            barrier, inc=1, device_id=nbr,
            device_id_type=pltpu.DeviceIdType.MESH,
        )
    pl.semaphore_wait(barrier, 3)

    y_rdma = pltpu.make_async_remote_copy(
        src_ref=strip_ref, dst_ref=ybuf,
        send_sem=y_send, recv_sem=y_recv,
        device_id=ynbr, device_id_type=pltpu.DeviceIdType.MESH,
    )
    y_rdma.start()
    y_rdma.wait()
    red = (
        strip_ref[...].astype(jnp.float32) + ybuf[...].astype(jnp.float32)
    )
    gather[p] = red.astype(jnp.bfloat16)

    copies = []

    def _emit(s, slot):
        if len(copies) >= 2:
            copies.pop(0)[1].wait()
        stage[slot] = gather[s].astype(jnp.float32)
        cp = pltpu.make_async_copy(
            stage.at[slot], out_ref.at[pl.ds(s * SM, SM)], copy_sem.at[slot]
        )
        cp.start()
        copies.append((slot, cp))

    _emit(p, 0)

    for r in range(FWD_ROUNDS):
        fwd = pltpu.make_async_remote_copy(
            src_ref=gather.at[(p - r) % P],
            dst_ref=gather.at[(p - r) % P],
            send_sem=fwd_send.at[r], recv_sem=fwd_recv.at[r],
            device_id=right, device_id_type=pltpu.DeviceIdType.MESH,
        )
        fwd.start()
        if r < BWD_ROUNDS:
            bwd = pltpu.make_async_remote_copy(
                src_ref=gather.at[(p + r) % P],
                dst_ref=gather.at[(p + r) % P],
                send_sem=bwd_send.at[r], recv_sem=bwd_recv.at[r],
                device_id=left, device_id_type=pltpu.DeviceIdType.MESH,
            )
            bwd.start()
        fwd.wait()
        _emit((p - 1 - r) % P, (2 * r + 1) % 2)
        if r < BWD_ROUNDS:
            bwd.wait()
            _emit((p + 1 + r) % P, (2 * r) % 2)

    while copies:
        copies.pop(0)[1].wait()


def _allreduce_gather(strip):
    return pl.pallas_call(
        _comm_body,
        out_shape=jax.ShapeDtypeStruct((M, N), jnp.float32),
        in_specs=[pl.BlockSpec(memory_space=pltpu.VMEM)],
        out_specs=pl.BlockSpec(memory_space=pltpu.MemorySpace.HBM),
        scratch_shapes=[
            pltpu.VMEM((SM, N), jnp.bfloat16),
            pltpu.VMEM((P, SM, N), jnp.bfloat16),
            pltpu.VMEM((2, SM, N), jnp.float32),
            pltpu.SemaphoreType.DMA,
            pltpu.SemaphoreType.DMA,
            pltpu.SemaphoreType.DMA((FWD_ROUNDS,)),
            pltpu.SemaphoreType.DMA((FWD_ROUNDS,)),
            pltpu.SemaphoreType.DMA((BWD_ROUNDS,)),
            pltpu.SemaphoreType.DMA((BWD_ROUNDS,)),
            pltpu.SemaphoreType.DMA((2,)),
        ],
        compiler_params=pltpu.CompilerParams(
            has_side_effects=True,
            collective_id=0,
            vmem_limit_bytes=62 * 1024 * 1024,
        ),
    )(strip)


def kernel(dy, W):
    my_x = lax.axis_index("x")
    my_z = lax.axis_index("z")
    p = jnp.where(my_x == 0, my_z, 7 - my_z)
    dy_strip = lax.dynamic_slice(dy, (p * SM, 0), (SM, K))
    strip = _strip_matmul(dy_strip, W)
    return _allreduce_gather(strip)


# device time: 351715 ns/iter; 2.4689x vs baseline; 1.0634x over previous
import jax
import jax.numpy as jnp
from jax import lax
from jax.experimental import pallas as pl
from jax.experimental.pallas import tpu as pltpu

M = 4096
N = 4096
K = 8192

P = 8
SM = M // P
BK = 256

FWD_ROUNDS = 4
BWD_ROUNDS = 4
HC = N // 2


def _ring_coords(q, my_y):
    xq = q // 4
    zq = jnp.where(xq == 0, q, 7 - q)
    return (xq, my_y, zq)


def _matmul_body(dy_ref, w_ref, out_ref, acc_ref):
    k = pl.program_id(0)

    @pl.when(k == 0)
    def _():
        acc_ref[...] = jnp.zeros_like(acc_ref)

    a = dy_ref[...].astype(jnp.bfloat16)
    b = w_ref[...].astype(jnp.bfloat16)
    acc_ref[...] += lax.dot_general(
        a, b, (((1,), (1,)), ((), ())), preferred_element_type=jnp.float32
    )

    @pl.when(k == pl.num_programs(0) - 1)
    def _():
        out_ref[...] = acc_ref[...].astype(jnp.bfloat16)


def _strip_matmul(dy_strip, w):
    return pl.pallas_call(
        _matmul_body,
        grid=(K // BK,),
        in_specs=[
            pl.BlockSpec((SM, BK), lambda k: (0, k)),
            pl.BlockSpec((N, BK), lambda k: (0, k)),
        ],
        out_specs=pl.BlockSpec((SM, N), lambda k: (0, 0)),
        out_shape=jax.ShapeDtypeStruct((SM, N), jnp.bfloat16),
        scratch_shapes=[pltpu.VMEM((SM, N), jnp.float32)],
        compiler_params=pltpu.CompilerParams(
            dimension_semantics=("arbitrary",),
            vmem_limit_bytes=60 * 1024 * 1024,
        ),
    )(dy_strip, w)


def _comm_body(
    strip_ref,
    out_ref,
    ybuf,
    gather,
    stage,
    y_send, y_recv,
    fwd_send, fwd_recv,
    bwd_send, bwd_recv,
    copy_sem,
):
    my_x = lax.axis_index("x")
    my_y = lax.axis_index("y")
    my_z = lax.axis_index("z")
    p = jnp.where(my_x == 0, my_z, 7 - my_z)
    right = _ring_coords((p + 1) % P, my_y)
    left = _ring_coords((p - 1) % P, my_y)
    ynbr = (my_x, 1 - my_y, my_z)

    barrier = pltpu.get_barrier_semaphore()
    for nbr in (ynbr, left, right):
        pl.semaphore_signal(
            barrier, inc=1, device_id=nbr,
            device_id_type=pltpu.DeviceIdType.MESH,
        )
    pl.semaphore_wait(barrier, 3)

    y_rdma = pltpu.make_async_remote_copy(
        src_ref=strip_ref, dst_ref=ybuf,
        send_sem=y_send, recv_sem=y_recv,
        device_id=ynbr, device_id_type=pltpu.DeviceIdType.MESH,
    )
    y_rdma.start()
    y_rdma.wait()
    red = (
        strip_ref[...].astype(jnp.float32) + ybuf[...].astype(jnp.float32)
    )
    gather[p] = red.astype(jnp.bfloat16)

    copies = []
    n_emitted = [0]

    def _emit(s):
        slot = n_emitted[0] % 2
        n_emitted[0] += 1
        if len(copies) >= 2:
            copies.pop(0).wait()
        stage[slot] = gather[s].astype(jnp.float32)
        cp = pltpu.make_async_copy(
            stage.at[slot], out_ref.at[pl.ds(s * SM, SM)], copy_sem.at[slot]
        )
        cp.start()
        copies.append(cp)

    _emit(p)

    for r in range(FWD_ROUNDS):
        if r < 3:
            fwd_src = gather.at[(p - r) % P]
            bwd_src = gather.at[(p + r) % P]
        else:
            fwd_src = gather.at[(p - r) % P, :, pl.ds(0, HC)]
            bwd_src = gather.at[(p + r) % P, :, pl.ds(HC, HC)]
        fwd = pltpu.make_async_remote_copy(
            src_ref=fwd_src, dst_ref=fwd_src,
            send_sem=fwd_send.at[r], recv_sem=fwd_recv.at[r],
            device_id=right, device_id_type=pltpu.DeviceIdType.MESH,
        )
        fwd.start()
        bwd = pltpu.make_async_remote_copy(
            src_ref=bwd_src, dst_ref=bwd_src,
            send_sem=bwd_send.at[r], recv_sem=bwd_recv.at[r],
            device_id=left, device_id_type=pltpu.DeviceIdType.MESH,
        )
        bwd.start()
        fwd.wait()
        bwd.wait()
        if r < 3:
            _emit((p - 1 - r) % P)
            _emit((p + 1 + r) % P)
        else:
            _emit((p + 4) % P)

    while copies:
        copies.pop(0).wait()


def _allreduce_gather(strip):
    return pl.pallas_call(
        _comm_body,
        out_shape=jax.ShapeDtypeStruct((M, N), jnp.float32),
        in_specs=[pl.BlockSpec(memory_space=pltpu.VMEM)],
        out_specs=pl.BlockSpec(memory_space=pltpu.MemorySpace.HBM),
        scratch_shapes=[
            pltpu.VMEM((SM, N), jnp.bfloat16),
            pltpu.VMEM((P, SM, N), jnp.bfloat16),
            pltpu.VMEM((2, SM, N), jnp.float32),
            pltpu.SemaphoreType.DMA,
            pltpu.SemaphoreType.DMA,
            pltpu.SemaphoreType.DMA((FWD_ROUNDS,)),
            pltpu.SemaphoreType.DMA((FWD_ROUNDS,)),
            pltpu.SemaphoreType.DMA((BWD_ROUNDS,)),
            pltpu.SemaphoreType.DMA((BWD_ROUNDS,)),
            pltpu.SemaphoreType.DMA((2,)),
        ],
        compiler_params=pltpu.CompilerParams(
            has_side_effects=True,
            collective_id=0,
            vmem_limit_bytes=62 * 1024 * 1024,
        ),
    )(strip)


def kernel(dy, W):
    my_x = lax.axis_index("x")
    my_z = lax.axis_index("z")
    p = jnp.where(my_x == 0, my_z, 7 - my_z)
    dy_strip = lax.dynamic_slice(dy, (p * SM, 0), (SM, K))
    strip = _strip_matmul(dy_strip, W)
    return _allreduce_gather(strip)


# device time: 337772 ns/iter; 2.5708x vs baseline; 1.0413x over previous
import jax
import jax.numpy as jnp
from jax import lax
from jax.experimental import pallas as pl
from jax.experimental.pallas import tpu as pltpu

M = 4096
N = 4096
K = 8192

P = 8
SM = M // P
BK = 256

FWD_ROUNDS = 4
BWD_ROUNDS = 4
HC = N // 2


def _ring_coords(q, my_y):
    xq = q // 4
    zq = jnp.where(xq == 0, q, 7 - q)
    return (xq, my_y, zq)


def _matmul_body(p_ref, dy_ref, w_ref, out_ref, acc_ref):
    k = pl.program_id(0)

    @pl.when(k == 0)
    def _():
        acc_ref[...] = jnp.zeros_like(acc_ref)

    a = dy_ref[...].astype(jnp.bfloat16)
    b = w_ref[...].astype(jnp.bfloat16)
    acc_ref[...] += lax.dot_general(
        a, b, (((1,), (1,)), ((), ())), preferred_element_type=jnp.float32
    )

    @pl.when(k == pl.num_programs(0) - 1)
    def _():
        out_ref[...] = acc_ref[...].astype(jnp.bfloat16)


def _strip_matmul(p, dy, w):
    return pl.pallas_call(
        _matmul_body,
        grid_spec=pltpu.PrefetchScalarGridSpec(
            num_scalar_prefetch=1,
            grid=(K // BK,),
            in_specs=[
                pl.BlockSpec((SM, BK), lambda k, pref: (pref[0], k)),
                pl.BlockSpec((N, BK), lambda k, pref: (0, k)),
            ],
            out_specs=pl.BlockSpec((SM, N), lambda k, pref: (0, 0)),
            scratch_shapes=[pltpu.VMEM((SM, N), jnp.float32)],
        ),
        out_shape=jax.ShapeDtypeStruct((SM, N), jnp.bfloat16),
        compiler_params=pltpu.CompilerParams(
            dimension_semantics=("arbitrary",),
            vmem_limit_bytes=60 * 1024 * 1024,
        ),
    )(p, dy, w)


def _comm_body(
    strip_ref,
    out_ref,
    ybuf,
    gather,
    stage,
    y_send, y_recv,
    fwd_send, fwd_recv,
    bwd_send, bwd_recv,
    copy_sem,
):
    my_x = lax.axis_index("x")
    my_y = lax.axis_index("y")
    my_z = lax.axis_index("z")
    p = jnp.where(my_x == 0, my_z, 7 - my_z)
    right = _ring_coords((p + 1) % P, my_y)
    left = _ring_coords((p - 1) % P, my_y)
    ynbr = (my_x, 1 - my_y, my_z)

    barrier = pltpu.get_barrier_semaphore()
    for nbr in (ynbr, left, right):
        pl.semaphore_signal(
            barrier, inc=1, device_id=nbr,
            device_id_type=pltpu.DeviceIdType.MESH,
        )
    pl.semaphore_wait(barrier, 3)

    y_rdma = pltpu.make_async_remote_copy(
        src_ref=strip_ref, dst_ref=ybuf,
        send_sem=y_send, recv_sem=y_recv,
        device_id=ynbr, device_id_type=pltpu.DeviceIdType.MESH,
    )
    y_rdma.start()
    y_rdma.wait()
    red = (
        strip_ref[...].astype(jnp.float32) + ybuf[...].astype(jnp.float32)
    )
    gather[p] = red.astype(jnp.bfloat16)

    copies = []
    n_emitted = [0]

    def _emit(s):
        slot = n_emitted[0] % 2
        n_emitted[0] += 1
        if len(copies) >= 2:
            copies.pop(0).wait()
        stage[slot] = gather[s].astype(jnp.float32)
        cp = pltpu.make_async_copy(
            stage.at[slot], out_ref.at[pl.ds(s * SM, SM)], copy_sem.at[slot]
        )
        cp.start()
        copies.append(cp)

    _emit(p)

    for r in range(FWD_ROUNDS):
        if r < 3:
            fwd_src = gather.at[(p - r) % P]
            bwd_src = gather.at[(p + r) % P]
        else:
            fwd_src = gather.at[(p - r) % P, :, pl.ds(0, HC)]
            bwd_src = gather.at[(p + r) % P, :, pl.ds(HC, HC)]
        fwd = pltpu.make_async_remote_copy(
            src_ref=fwd_src, dst_ref=fwd_src,
            send_sem=fwd_send.at[r], recv_sem=fwd_recv.at[r],
            device_id=right, device_id_type=pltpu.DeviceIdType.MESH,
        )
        fwd.start()
        bwd = pltpu.make_async_remote_copy(
            src_ref=bwd_src, dst_ref=bwd_src,
            send_sem=bwd_send.at[r], recv_sem=bwd_recv.at[r],
            device_id=left, device_id_type=pltpu.DeviceIdType.MESH,
        )
        bwd.start()
        fwd.wait()
        bwd.wait()
        if r < 3:
            _emit((p - 1 - r) % P)
            _emit((p + 1 + r) % P)
        else:
            _emit((p + 4) % P)

    while copies:
        copies.pop(0).wait()


def _allreduce_gather(strip):
    return pl.pallas_call(
        _comm_body,
        out_shape=jax.ShapeDtypeStruct((M, N), jnp.float32),
        in_specs=[pl.BlockSpec(memory_space=pltpu.VMEM)],
        out_specs=pl.BlockSpec(memory_space=pltpu.MemorySpace.HBM),
        scratch_shapes=[
            pltpu.VMEM((SM, N), jnp.bfloat16),
            pltpu.VMEM((P, SM, N), jnp.bfloat16),
            pltpu.VMEM((2, SM, N), jnp.float32),
            pltpu.SemaphoreType.DMA,
            pltpu.SemaphoreType.DMA,
            pltpu.SemaphoreType.DMA((FWD_ROUNDS,)),
            pltpu.SemaphoreType.DMA((FWD_ROUNDS,)),
            pltpu.SemaphoreType.DMA((BWD_ROUNDS,)),
            pltpu.SemaphoreType.DMA((BWD_ROUNDS,)),
            pltpu.SemaphoreType.DMA((2,)),
        ],
        compiler_params=pltpu.CompilerParams(
            has_side_effects=True,
            collective_id=0,
            vmem_limit_bytes=62 * 1024 * 1024,
        ),
    )(strip)


def kernel(dy, W):
    my_x = lax.axis_index("x")
    my_z = lax.axis_index("z")
    p = jnp.where(my_x == 0, my_z, 7 - my_z)
    strip = _strip_matmul(p.astype(jnp.int32).reshape(1), dy, W)
    return _allreduce_gather(strip)


# device time: 317155 ns/iter; 2.7380x vs baseline; 1.0650x over previous
import jax
import jax.numpy as jnp
from jax import lax
from jax.experimental import pallas as pl
from jax.experimental.pallas import tpu as pltpu

M = 4096
N = 4096
K = 8192

P = 8
SM = M // P
BK = 256

FWD_ROUNDS = 4
BWD_ROUNDS = 4

CA = 1280
CO = N - 2 * CA
CL = CA + CO
CH = CL // 2


def _ring_coords(q, my_y):
    xq = q // 4
    zq = jnp.where(xq == 0, q, 7 - q)
    return (xq, my_y, zq)


def _matmul_body(p_ref, dy_ref, w_ref, out_ref, acc_ref):
    k = pl.program_id(0)

    @pl.when(k == 0)
    def _():
        acc_ref[...] = jnp.zeros_like(acc_ref)

    a = dy_ref[...].astype(jnp.bfloat16)
    b = w_ref[...].astype(jnp.bfloat16)
    acc_ref[...] += lax.dot_general(
        a, b, (((1,), (1,)), ((), ())), preferred_element_type=jnp.float32
    )

    @pl.when(k == pl.num_programs(0) - 1)
    def _():
        out_ref[...] = acc_ref[...].astype(jnp.bfloat16)


def _strip_matmul(p, dy, w):
    return pl.pallas_call(
        _matmul_body,
        grid_spec=pltpu.PrefetchScalarGridSpec(
            num_scalar_prefetch=1,
            grid=(K // BK,),
            in_specs=[
                pl.BlockSpec((SM, BK), lambda k, pref: (pref[0], k)),
                pl.BlockSpec((N, BK), lambda k, pref: (0, k)),
            ],
            out_specs=pl.BlockSpec((SM, N), lambda k, pref: (0, 0)),
            scratch_shapes=[pltpu.VMEM((SM, N), jnp.float32)],
        ),
        out_shape=jax.ShapeDtypeStruct((SM, N), jnp.bfloat16),
        compiler_params=pltpu.CompilerParams(
            dimension_semantics=("arbitrary",),
            vmem_limit_bytes=60 * 1024 * 1024,
        ),
    )(p, dy, w)


def _comm_body(
    strip_ref,
    out_ref,
    ybuf,
    gather,
    stage,
    y_send, y_recv,
    fwd_send, fwd_recv,
    bwd_send, bwd_recv,
    rung_send, rung_recv,
    copy_sem,
):
    my_x = lax.axis_index("x")
    my_y = lax.axis_index("y")
    my_z = lax.axis_index("z")
    p = jnp.where(my_x == 0, my_z, 7 - my_z)
    right = _ring_coords((p + 1) % P, my_y)
    left = _ring_coords((p - 1) % P, my_y)
    ynbr = (my_x, 1 - my_y, my_z)

    barrier = pltpu.get_barrier_semaphore()
    for nbr in (ynbr, left, right):
        pl.semaphore_signal(
            barrier, inc=1, device_id=nbr,
            device_id_type=pltpu.DeviceIdType.MESH,
        )
    pl.semaphore_wait(barrier, 3)

    y_rdma = pltpu.make_async_remote_copy(
        src_ref=strip_ref, dst_ref=ybuf,
        send_sem=y_send, recv_sem=y_recv,
        device_id=ynbr, device_id_type=pltpu.DeviceIdType.MESH,
    )
    y_rdma.start()
    y_rdma.wait()
    red = (
        strip_ref[...].astype(jnp.float32) + ybuf[...].astype(jnp.float32)
    )
    gather[p] = red.astype(jnp.bfloat16)

    for yv in (0, 1):

        @pl.when(my_y == yv)
        def _(yv=yv):
            ro = CA * yv
            rs = CL * yv

            copies = []
            n_emitted = [0]

            def _emit(s):
                slot = n_emitted[0] % 2
                n_emitted[0] += 1
                if len(copies) >= 2:
                    copies.pop(0).wait()
                stage[slot] = gather[s].astype(jnp.float32)
                cp = pltpu.make_async_copy(
                    stage.at[slot],
                    out_ref.at[pl.ds(s * SM, SM)],
                    copy_sem.at[slot],
                )
                cp.start()
                copies.append(cp)

            def _rung(s, o):
                return pltpu.make_async_remote_copy(
                    src_ref=gather.at[s, :, pl.ds(rs, CA)],
                    dst_ref=gather.at[s, :, pl.ds(rs, CA)],
                    send_sem=rung_send.at[o], recv_sem=rung_recv.at[o],
                    device_id=ynbr, device_id_type=pltpu.DeviceIdType.MESH,
                )

            _emit(p)

            rungs = []
            for r in range(FWD_ROUNDS):
                sf = (p - r) % P
                sb = (p + r) % P
                if r < 3:
                    fwd_src = gather.at[sf, :, pl.ds(ro, CL)]
                    bwd_src = gather.at[sb, :, pl.ds(ro, CL)]
                else:
                    fwd_src = gather.at[sf, :, pl.ds(ro, CH)]
                    bwd_src = gather.at[sb, :, pl.ds(ro + CH, CH)]
                fwd = pltpu.make_async_remote_copy(
                    src_ref=fwd_src, dst_ref=fwd_src,
                    send_sem=fwd_send.at[r], recv_sem=fwd_recv.at[r],
                    device_id=right, device_id_type=pltpu.DeviceIdType.MESH,
                )
                fwd.start()
                bwd = pltpu.make_async_remote_copy(
                    src_ref=bwd_src, dst_ref=bwd_src,
                    send_sem=bwd_send.at[r], recv_sem=bwd_recv.at[r],
                    device_id=left, device_id_type=pltpu.DeviceIdType.MESH,
                )
                bwd.start()
                if r >= 1:
                    for o, s in ((2 * (r - 1), sf), (2 * (r - 1) + 1, sb)):
                        rg = _rung(s, o)
                        rg.start()
                        rungs.append(rg)
                fwd.wait()
                bwd.wait()
                if r >= 1:
                    rungs.pop(0).wait()
                    rungs.pop(0).wait()
                    _emit(sf)
                    _emit(sb)
            anti = (p + 4) % P
            rg = _rung(anti, 6)
            rg.start()
            rg.wait()
            _emit(anti)
            while copies:
                copies.pop(0).wait()


def _allreduce_gather(strip):
    return pl.pallas_call(
        _comm_body,
        out_shape=jax.ShapeDtypeStruct((M, N), jnp.float32),
        in_specs=[pl.BlockSpec(memory_space=pltpu.VMEM)],
        out_specs=pl.BlockSpec(memory_space=pltpu.MemorySpace.HBM),
        scratch_shapes=[
            pltpu.VMEM((SM, N), jnp.bfloat16),
            pltpu.VMEM((P, SM, N), jnp.bfloat16),
            pltpu.VMEM((2, SM, N), jnp.float32),
            pltpu.SemaphoreType.DMA,
            pltpu.SemaphoreType.DMA,
            pltpu.SemaphoreType.DMA((FWD_ROUNDS,)),
            pltpu.SemaphoreType.DMA((FWD_ROUNDS,)),
            pltpu.SemaphoreType.DMA((BWD_ROUNDS,)),
            pltpu.SemaphoreType.DMA((BWD_ROUNDS,)),
            pltpu.SemaphoreType.DMA((7,)),
            pltpu.SemaphoreType.DMA((7,)),
            pltpu.SemaphoreType.DMA((2,)),
        ],
        compiler_params=pltpu.CompilerParams(
            has_side_effects=True,
            collective_id=0,
            vmem_limit_bytes=62 * 1024 * 1024,
        ),
    )(strip)


def kernel(dy, W):
    my_x = lax.axis_index("x")
    my_z = lax.axis_index("z")
    p = jnp.where(my_x == 0, my_z, 7 - my_z)
    strip = _strip_matmul(p.astype(jnp.int32).reshape(1), dy, W)
    return _allreduce_gather(strip)
